# baseline (speedup 1.0000x reference)
import functools

import jax
import jax.numpy as jnp
from jax import lax
from jax.experimental import pallas as pl
from jax.experimental.pallas import tpu as pltpu
from jax.experimental.pallas import tpu_sc as plsc

N = 10000
D = 128
NC = 2
NS = 16
NW = NC * NS
CHUNK = 128
LANES = 3
N_ACC = 10112
BLK = 2000


def _prep_body(f_ref, n_ref, h_ref):
    h_ref[...] = f_ref[...] * n_ref[...]


def _prep(feature, norm):
    return pl.pallas_call(
        _prep_body,
        grid=(N // BLK,),
        in_specs=[
            pl.BlockSpec((BLK, D), lambda i: (i, 0)),
            pl.BlockSpec((BLK, 1), lambda i: (i, 0)),
        ],
        out_specs=pl.BlockSpec((BLK, D), lambda i: (i, 0)),
        out_shape=jax.ShapeDtypeStruct((N, D), jnp.float32),
    )(feature, norm)


def _final_body(p_ref, n_ref, w_ref, b_ref, o_ref):
    acc = (p_ref[0] + p_ref[1]) * n_ref[...]
    o_ref[...] = lax.dot_general(
        acc, w_ref[...], (((1,), (1,)), ((), ())),
        preferred_element_type=jnp.float32) + b_ref[...]


def _final(partials, norm, W, b2):
    return pl.pallas_call(
        _final_body,
        grid=(N // BLK,),
        in_specs=[
            pl.BlockSpec((2, BLK, D), lambda i: (0, i, 0)),
            pl.BlockSpec((BLK, 1), lambda i: (i, 0)),
            pl.BlockSpec((D, D), lambda i: (0, 0)),
            pl.BlockSpec((1, D), lambda i: (0, 0)),
        ],
        out_specs=pl.BlockSpec((BLK, D), lambda i: (i, 0)),
        out_shape=jax.ShapeDtypeStruct((N, D), jnp.float32),
    )(partials, norm, W, b2)


def _sc_segment_sum(h, src_p, dst_p, cpw):
    mesh = plsc.VectorSubcoreMesh(
        core_axis_name="c", subcore_axis_name="s",
        num_cores=NC, num_subcores=NS)

    @functools.partial(
        pl.kernel,
        out_type=jax.ShapeDtypeStruct((NC, N_ACC, D), jnp.float32),
        mesh=mesh,
        scratch_types=[
            [pltpu.VMEM((CHUNK,), jnp.int32)] * LANES,
            [pltpu.VMEM((CHUNK,), jnp.int32)] * LANES,
            pltpu.VMEM((LANES, CHUNK, D), jnp.float32),
            pltpu.VMEM_SHARED((N_ACC, D), jnp.float32),
            [pltpu.SemaphoreType.DMA] * LANES,
            [pltpu.SemaphoreType.DMA] * LANES,
        ],
    )
    def k(h_hbm, src_hbm, dst_hbm, out_hbm, sidx, didx, rows, accum,
          isem, gsem):
        c = lax.axis_index("c")
        s = lax.axis_index("s")
        w = c * NS + s

        def zb(i, carry):
            rows[0, i // 8, pl.ds((i % 8) * 16, 16)] = (
                jnp.zeros((16,), jnp.float32))
            return carry
        lax.fori_loop(0, CHUNK * (D // 16), zb, 0)
        slab = N_ACC // NS
        for j in range(slab // CHUNK):
            pltpu.sync_copy(rows.at[0],
                            accum.at[pl.ds(s * slab + j * CHUNK, CHUNK)])
        rem = slab % CHUNK
        if rem:
            pltpu.sync_copy(
                rows.at[0, pl.ds(0, rem)],
                accum.at[pl.ds(s * slab + (slab // CHUNK) * CHUNK, rem)])
        plsc.subcore_barrier()

        base = w * cpw * CHUNK

        def idx_descs(j, b):
            off = base + j * CHUNK
            return (
                pltpu.make_async_copy(
                    src_hbm.at[pl.ds(off, CHUNK)], sidx[b], isem[b]),
                pltpu.make_async_copy(
                    dst_hbm.at[pl.ds(off, CHUNK)], didx[b], isem[b]),
            )

        def gather_desc(b):
            return pltpu.make_async_copy(h_hbm.at[sidx[b]], rows.at[b],
                                         gsem[b])

        for d in idx_descs(0, 0):
            d.start()

        def body(jo, carry):
            for b in range(LANES):
                j = jo * LANES + b
                m = (b + 1) % LANES

                @pl.when(j < cpw)
                def _():
                    for d in idx_descs(j, b):
                        d.wait()
                    gather_desc(b).start()

                @pl.when((j >= 2) & (j <= cpw + 1))
                def _():
                    gather_desc(m).wait()
                    pltpu.sync_copy(rows.at[m], accum.at[didx[m]], add=True)

                @pl.when(j + 1 < cpw)
                def _():
                    for d in idx_descs(j + 1, m):
                        d.start()
            return carry
        lax.fori_loop(0, cpw // LANES + 1, body, 0)

        plsc.subcore_barrier()
        pltpu.sync_copy(accum.at[pl.ds(s * slab, slab)],
                        out_hbm.at[c, pl.ds(s * slab, slab)])

    return k(h, src_p, dst_p)[:, :N, :]


def kernel(feature, edge_index, norm, W, b):
    E = edge_index.shape[1]
    cpw = -(-E // (NW * CHUNK))
    cpw = -(-cpw // LANES) * LANES
    e_pad = NW * cpw * CHUNK
    src = edge_index[0].astype(jnp.int32)
    dst = edge_index[1].astype(jnp.int32)
    src_p = jnp.concatenate([src, jnp.zeros((e_pad - E,), jnp.int32)])
    dst_p = jnp.concatenate([dst, jnp.full((e_pad - E,), N, jnp.int32)])

    h = _prep(feature, norm)
    partials = _sc_segment_sum(h, src_p, dst_p, cpw)
    return _final(partials, norm, W, b.reshape(1, D))


# speedup vs baseline: 1.4573x; 1.4573x over previous
import functools

import jax
import jax.numpy as jnp
from jax import lax
from jax.experimental import pallas as pl
from jax.experimental.pallas import tpu as pltpu
from jax.experimental.pallas import tpu_sc as plsc

N = 10000
D = 128
NC = 2
NS = 16
NW = NC * NS
CHUNK = 128
PH = 2
N_ACC = 10240
BLK = 2000


def _prep_body(f_ref, n_ref, h_ref):
    h_ref[...] = f_ref[...] * n_ref[...]


def _prep(feature, norm):
    return pl.pallas_call(
        _prep_body,
        grid=(N // BLK,),
        in_specs=[
            pl.BlockSpec((BLK, D), lambda i: (i, 0)),
            pl.BlockSpec((BLK, 1), lambda i: (i, 0)),
        ],
        out_specs=pl.BlockSpec((BLK, D), lambda i: (i, 0)),
        out_shape=jax.ShapeDtypeStruct((N, D), jnp.float32),
    )(feature, norm)


def _final_body(p_ref, n_ref, w_ref, b_ref, o_ref):
    acc = (p_ref[0] + p_ref[1]) * n_ref[...]
    o_ref[...] = lax.dot_general(
        acc, w_ref[...], (((1,), (1,)), ((), ())),
        preferred_element_type=jnp.float32) + b_ref[...]


def _final(partials, norm, W, b2):
    return pl.pallas_call(
        _final_body,
        grid=(N // BLK,),
        in_specs=[
            pl.BlockSpec((2, BLK, D), lambda i: (0, i, 0)),
            pl.BlockSpec((BLK, 1), lambda i: (i, 0)),
            pl.BlockSpec((D, D), lambda i: (0, 0)),
            pl.BlockSpec((1, D), lambda i: (0, 0)),
        ],
        out_specs=pl.BlockSpec((BLK, D), lambda i: (i, 0)),
        out_shape=jax.ShapeDtypeStruct((N, D), jnp.float32),
    )(partials, norm, W, b2)


def _sc_segment_sum(h, src_p, dst_p, cpw):
    mesh = plsc.VectorSubcoreMesh(
        core_axis_name="c", subcore_axis_name="s",
        num_cores=NC, num_subcores=NS)

    @functools.partial(
        pl.kernel,
        out_type=jax.ShapeDtypeStruct((NC, N_ACC, D), jnp.float32),
        mesh=mesh,
        scratch_types=[
            pltpu.VMEM((CHUNK,), jnp.int32),
            pltpu.VMEM((CHUNK,), jnp.int32),
            pltpu.VMEM((CHUNK,), jnp.int32),
            pltpu.VMEM((CHUNK,), jnp.int32),
            pltpu.VMEM((2, CHUNK, D), jnp.float32),
            pltpu.VMEM_SHARED((N_ACC, D), jnp.float32),
            pltpu.SemaphoreType.DMA,
            pltpu.SemaphoreType.DMA,
            pltpu.SemaphoreType.DMA,
            pltpu.SemaphoreType.DMA,
        ],
    )
    def k(h_hbm, src_hbm, dst_hbm, out_hbm, sidx0, sidx1, didx0, didx1,
          rows, accum, isem0, isem1, gsem0, gsem1):
        c = lax.axis_index("c")
        s = lax.axis_index("s")
        w = c * NS + s
        sidx = (sidx0, sidx1)
        didx = (didx0, didx1)
        isem = (isem0, isem1)
        gsem = (gsem0, gsem1)

        def zb(i, carry):
            rows[0, i // 8, pl.ds((i % 8) * 16, 16)] = (
                jnp.zeros((16,), jnp.float32))
            return carry
        lax.fori_loop(0, CHUNK * (D // 16), zb, 0)
        slab = N_ACC // NS
        for j in range(slab // CHUNK):
            pltpu.sync_copy(rows.at[0],
                            accum.at[pl.ds(s * slab + j * CHUNK, CHUNK)])
        plsc.subcore_barrier()

        base = w * cpw * CHUNK

        def idx_descs(j, b):
            off = base + j * CHUNK
            return (
                pltpu.make_async_copy(
                    src_hbm.at[pl.ds(off, CHUNK)], sidx[b], isem[b]),
                pltpu.make_async_copy(
                    dst_hbm.at[pl.ds(off, CHUNK)], didx[b], isem[b]),
            )

        def gather_desc(b):
            return pltpu.make_async_copy(h_hbm.at[sidx[b]], rows.at[b],
                                         gsem[b])

        for d in idx_descs(0, 0):
            d.start()

        def body(jo, carry):
            for b in range(2):
                j = jo * 2 + b

                @pl.when(j < cpw)
                def _():
                    for d in idx_descs(j, b):
                        d.wait()
                    gather_desc(b).start()

                @pl.when((j >= 1) & (j <= cpw))
                def _():
                    gather_desc(1 - b).wait()
                    pltpu.sync_copy(rows.at[1 - b],
                                    accum.at[didx[1 - b]], add=True)

                @pl.when(j + 1 < cpw)
                def _():
                    for d in idx_descs(j + 1, 1 - b):
                        d.start()
            return carry
        lax.fori_loop(0, cpw // 2 + 1, body, 0)

        plsc.subcore_barrier()
        pltpu.sync_copy(accum.at[pl.ds(s * slab, slab)],
                        out_hbm.at[c, pl.ds(s * slab, slab)])

    return k(h, src_p, dst_p)


def kernel(feature, edge_index, norm, W, b):
    E = edge_index.shape[1]
    cpw = -(-E // (NW * CHUNK))
    cpw = -(-cpw // (2 * PH)) * (2 * PH)
    e_pad = NW * cpw * CHUNK
    src = edge_index[0].astype(jnp.int32)
    dst = edge_index[1].astype(jnp.int32)
    src_p = jnp.concatenate([src, jnp.zeros((e_pad - E,), jnp.int32)])
    dst_p = jnp.concatenate([dst, jnp.full((e_pad - E,), N, jnp.int32)])

    h = _prep(feature, norm)
    partials = _sc_segment_sum(h, src_p, dst_p, cpw)
    return _final(partials, norm, W, b.reshape(1, D))


# speedup vs baseline: 1.4573x; 1.0000x over previous
import functools

import jax
import jax.numpy as jnp
from jax import lax
from jax.experimental import pallas as pl
from jax.experimental.pallas import tpu as pltpu
from jax.experimental.pallas import tpu_sc as plsc

N = 10000
D = 128
NC = 2
NS = 16
NW = NC * NS
CHUNK = 128
N_ACC = 10240
BLK = 2000


def _prep_body(f_ref, n_ref, h_ref):
    h_ref[...] = f_ref[...] * n_ref[...]


def _prep(feature, norm):
    return pl.pallas_call(
        _prep_body,
        grid=(N // BLK,),
        in_specs=[
            pl.BlockSpec((BLK, D), lambda i: (i, 0)),
            pl.BlockSpec((BLK, 1), lambda i: (i, 0)),
        ],
        out_specs=pl.BlockSpec((BLK, D), lambda i: (i, 0)),
        out_shape=jax.ShapeDtypeStruct((N, D), jnp.float32),
    )(feature, norm)


def _final_body(p_ref, n_ref, w_ref, b_ref, o_ref):
    acc = (p_ref[0] + p_ref[1]) * n_ref[...]
    o_ref[...] = lax.dot_general(
        acc, w_ref[...], (((1,), (1,)), ((), ())),
        preferred_element_type=jnp.float32) + b_ref[...]


def _final(partials, norm, W, b2):
    return pl.pallas_call(
        _final_body,
        grid=(N // BLK,),
        in_specs=[
            pl.BlockSpec((2, BLK, D), lambda i: (0, i, 0)),
            pl.BlockSpec((BLK, 1), lambda i: (i, 0)),
            pl.BlockSpec((D, D), lambda i: (0, 0)),
            pl.BlockSpec((1, D), lambda i: (0, 0)),
        ],
        out_specs=pl.BlockSpec((BLK, D), lambda i: (i, 0)),
        out_shape=jax.ShapeDtypeStruct((N, D), jnp.float32),
    )(partials, norm, W, b2)


def _sc_segment_sum(h, src_p, dst_p, cpw):
    mesh = plsc.VectorSubcoreMesh(
        core_axis_name="c", subcore_axis_name="s",
        num_cores=NC, num_subcores=NS)

    @functools.partial(
        pl.kernel,
        out_type=jax.ShapeDtypeStruct((NC, N_ACC, D), jnp.float32),
        mesh=mesh,
        scratch_types=[
            pltpu.VMEM((CHUNK,), jnp.int32),
            pltpu.VMEM((CHUNK,), jnp.int32),
            pltpu.VMEM((CHUNK,), jnp.int32),
            pltpu.VMEM((CHUNK,), jnp.int32),
            pltpu.VMEM((2, CHUNK, D), jnp.float32),
            pltpu.VMEM_SHARED((N_ACC, D), jnp.float32),
            pltpu.SemaphoreType.DMA,
            pltpu.SemaphoreType.DMA,
            pltpu.SemaphoreType.DMA,
            pltpu.SemaphoreType.DMA,
        ],
    )
    def k(h_hbm, src_hbm, dst_hbm, out_hbm, sidx0, sidx1, didx0, didx1,
          rows, accum, isem0, isem1, gsem0, gsem1):
        c = lax.axis_index("c")
        s = lax.axis_index("s")
        w = c * NS + s
        sidx = (sidx0, sidx1)
        didx = (didx0, didx1)
        isem = (isem0, isem1)
        gsem = (gsem0, gsem1)

        def zb(i, carry):
            rows[0, i // 8, pl.ds((i % 8) * 16, 16)] = (
                jnp.zeros((16,), jnp.float32))
            return carry
        lax.fori_loop(0, CHUNK * (D // 16), zb, 0)
        slab = N_ACC // NS
        for j in range(slab // CHUNK):
            pltpu.sync_copy(rows.at[0],
                            accum.at[pl.ds(s * slab + j * CHUNK, CHUNK)])
        plsc.subcore_barrier()

        base = w * cpw * CHUNK

        def idx_descs(j, b):
            off = base + j * CHUNK
            return (
                pltpu.make_async_copy(
                    src_hbm.at[pl.ds(off, CHUNK)], sidx[b], isem[b]),
                pltpu.make_async_copy(
                    dst_hbm.at[pl.ds(off, CHUNK)], didx[b], isem[b]),
            )

        def gather_desc(b):
            return pltpu.make_async_copy(h_hbm.at[sidx[b]], rows.at[b],
                                         gsem[b])

        for d in idx_descs(0, 0):
            d.start()

        def body(jo, carry):
            for b in range(2):
                j = jo * 2 + b

                @pl.when(j < cpw)
                def _():
                    for d in idx_descs(j, b):
                        d.wait()
                    gather_desc(b).start()

                @pl.when((j >= 1) & (j <= cpw))
                def _():
                    gather_desc(1 - b).wait()
                    pltpu.sync_copy(rows.at[1 - b],
                                    accum.at[didx[1 - b]], add=True)

                @pl.when(j + 1 < cpw)
                def _():
                    for d in idx_descs(j + 1, 1 - b):
                        d.start()
            return carry
        lax.fori_loop(0, cpw // 2 + 1, body, 0)

        plsc.subcore_barrier()
        pltpu.sync_copy(accum.at[pl.ds(s * slab, slab)],
                        out_hbm.at[c, pl.ds(s * slab, slab)])

    return k(h, src_p, dst_p)


def kernel(feature, edge_index, norm, W, b):
    E = edge_index.shape[1]
    cpw = -(-E // (NW * CHUNK))
    cpw = -(-cpw // 4) * 4
    e_pad = NW * cpw * CHUNK
    src = edge_index[0].astype(jnp.int32)
    dst = edge_index[1].astype(jnp.int32)
    src_p = jnp.concatenate([src, jnp.zeros((e_pad - E,), jnp.int32)])
    dst_p = jnp.concatenate([dst, jnp.full((e_pad - E,), N, jnp.int32)])

    h = _prep(feature, norm)
    partials = _sc_segment_sum(h, src_p, dst_p, cpw)
    return _final(partials, norm, W, b.reshape(1, D))


# speedup vs baseline: 1.6563x; 1.1366x over previous
import functools

import jax
import jax.numpy as jnp
from jax import lax
from jax.experimental import pallas as pl
from jax.experimental.pallas import tpu as pltpu
from jax.experimental.pallas import tpu_sc as plsc

N = 10000
D = 128
NC = 2
NS = 16
NW = NC * NS
CHUNK = 128
N_ACC = 10240
BLK = 2000


def _prep_body(f_ref, n_ref, h_ref):
    h_ref[...] = f_ref[...] * n_ref[...]


def _prep(feature, norm):
    return pl.pallas_call(
        _prep_body,
        grid=(N // BLK,),
        in_specs=[
            pl.BlockSpec((BLK, D), lambda i: (i, 0)),
            pl.BlockSpec((BLK, 1), lambda i: (i, 0)),
        ],
        out_specs=pl.BlockSpec((BLK, D), lambda i: (i, 0)),
        out_shape=jax.ShapeDtypeStruct((N, D), jnp.float32),
    )(feature, norm)


def _final_body(p_ref, n_ref, w_ref, b_ref, o_ref):
    acc = (p_ref[0] + p_ref[1]) * n_ref[...]
    o_ref[...] = lax.dot_general(
        acc, w_ref[...], (((1,), (1,)), ((), ())),
        preferred_element_type=jnp.float32) + b_ref[...]


def _final(partials, norm, W, b2):
    return pl.pallas_call(
        _final_body,
        grid=(N // BLK,),
        in_specs=[
            pl.BlockSpec((2, BLK, D), lambda i: (0, i, 0)),
            pl.BlockSpec((BLK, 1), lambda i: (i, 0)),
            pl.BlockSpec((D, D), lambda i: (0, 0)),
            pl.BlockSpec((1, D), lambda i: (0, 0)),
        ],
        out_specs=pl.BlockSpec((BLK, D), lambda i: (i, 0)),
        out_shape=jax.ShapeDtypeStruct((N, D), jnp.float32),
    )(partials, norm, W, b2)


def _sc_segment_sum(h, src_p, dst_p, cpw):
    mesh = plsc.VectorSubcoreMesh(
        core_axis_name="c", subcore_axis_name="s",
        num_cores=NC, num_subcores=NS)

    @functools.partial(
        pl.kernel,
        out_type=jax.ShapeDtypeStruct((NC, N_ACC, D), jnp.float32),
        mesh=mesh,
        scratch_types=[
            pltpu.VMEM((CHUNK,), jnp.int32),
            pltpu.VMEM((CHUNK,), jnp.int32),
            pltpu.VMEM((CHUNK,), jnp.int32),
            pltpu.VMEM((CHUNK,), jnp.int32),
            pltpu.VMEM((2, CHUNK, D), jnp.float32),
            pltpu.VMEM_SHARED((N_ACC, D), jnp.float32),
            pltpu.SemaphoreType.DMA,
            pltpu.SemaphoreType.DMA,
            pltpu.SemaphoreType.DMA,
            pltpu.SemaphoreType.DMA,
        ],
    )
    def k(h_hbm, src_hbm, dst_hbm, out_hbm, sidx0, sidx1, didx0, didx1,
          rows, accum, isem0, isem1, gsem0, gsem1):
        c = lax.axis_index("c")
        s = lax.axis_index("s")
        w = c * NS + s
        sidx = (sidx0, sidx1)
        didx = (didx0, didx1)
        isem = (isem0, isem1)
        gsem = (gsem0, gsem1)

        def zb(i, carry):
            rows[0, i // 8, pl.ds((i % 8) * 16, 16)] = (
                jnp.zeros((16,), jnp.float32))
            return carry
        lax.fori_loop(0, CHUNK * (D // 16), zb, 0)
        slab = N_ACC // NS
        for j in range(slab // CHUNK):
            pltpu.sync_copy(rows.at[0],
                            accum.at[pl.ds(s * slab + j * CHUNK, CHUNK)])
        plsc.subcore_barrier()

        base = w * cpw * CHUNK

        def idx_descs(j, b):
            off = base + j * CHUNK
            return (
                pltpu.make_async_copy(
                    src_hbm.at[pl.ds(off, CHUNK)], sidx[b], isem[b]),
                pltpu.make_async_copy(
                    dst_hbm.at[pl.ds(off, CHUNK)], didx[b], isem[b]),
            )

        def gather_desc(b):
            return pltpu.make_async_copy(h_hbm.at[sidx[b]], rows.at[b],
                                         gsem[b])

        for d in idx_descs(0, 0):
            d.start()

        def body(jo, carry):
            for b in range(2):
                j = jo * 2 + b

                @pl.when(j < cpw)
                def _():
                    for d in idx_descs(j, b):
                        d.wait()
                    gather_desc(b).start()

                @pl.when((j >= 1) & (j <= cpw))
                def _():
                    gather_desc(1 - b).wait()
                    pltpu.sync_copy(rows.at[1 - b],
                                    accum.at[didx[1 - b]], add=True)

                @pl.when(j + 1 < cpw)
                def _():
                    for d in idx_descs(j + 1, 1 - b):
                        d.start()
            return carry
        lax.fori_loop(0, cpw // 2 + 1, body, 0)

        plsc.subcore_barrier()
        pltpu.sync_copy(accum.at[pl.ds(s * slab, slab)],
                        out_hbm.at[c, pl.ds(s * slab, slab)])

    return k(h, src_p, dst_p)


def kernel(feature, edge_index, norm, W, b):
    E = edge_index.shape[1]
    cpw = -(-E // (NW * CHUNK))
    cpw = -(-cpw // 4) * 4
    e_pad = NW * cpw * CHUNK
    src = edge_index[0].astype(jnp.int32)
    dst = edge_index[1].astype(jnp.int32)
    if E % NW == 0:
        epw = E // NW
        src_p = jnp.pad(src.reshape(NW, epw),
                        ((0, 0), (0, cpw * CHUNK - epw))).reshape(-1)
        dst_p = jnp.pad(dst.reshape(NW, epw),
                        ((0, 0), (0, cpw * CHUNK - epw)),
                        constant_values=N).reshape(-1)
    else:
        src_p = jnp.concatenate([src, jnp.zeros((e_pad - E,), jnp.int32)])
        dst_p = jnp.concatenate([dst, jnp.full((e_pad - E,), N, jnp.int32)])

    h = _prep(feature, norm)
    partials = _sc_segment_sum(h, src_p, dst_p, cpw)
    return _final(partials, norm, W, b.reshape(1, D))
